# Initial kernel scaffold; baseline (speedup 1.0000x reference)
import jax
import jax.numpy as jnp
from jax.experimental import pallas as pl


def kernel(x, edge_index, batch, W0, b0, g0, be0, W1, b1, g1, be1, Wh0, bh0, gh0, beh0, Wh1, bh1, gh1, beh1, Wf, bf):
    raise NotImplementedError("write your pallas kernel here")


# speedup vs baseline: 7.8138x; 7.8138x over previous
import functools

import jax
import jax.numpy as jnp
from jax import lax
from jax.experimental import pallas as pl
from jax.experimental.pallas import tpu as pltpu
from jax.experimental.pallas import tpu_sc as plsc

N = 10000
E = 320000
G = 64
D = 128

NW = 32
CH = 128
CPW = 80
EPW = CPW * CH
E_PAD = NW * EPW
GRP = 8
N_ACC = 10240
RPS = N_ACC // 16
BN_EPS = 1e-5


_KCH = RPS // CH


@functools.cache
def _make_sc_degree():
    mesh = plsc.VectorSubcoreMesh(core_axis_name="core",
                                  subcore_axis_name="subcore")
    return functools.partial(
        pl.kernel,
        out_type=jax.ShapeDtypeStruct((2, N_ACC, 16), jnp.float32),
        mesh=mesh,
        scratch_types=[
            pltpu.VMEM((CPW, CH), jnp.int32),
            pltpu.VMEM((_KCH, CH), jnp.int32),
            pltpu.VMEM((CH, 16), jnp.float32),
            pltpu.VMEM((CH, 16), jnp.float32),
            pltpu.VMEM_SHARED((N_ACC, 16), jnp.float32),
        ],
    )(_sc_degree_body)


def _sc_degree_body(dst_hbm, eye_hbm, degp_hbm, didx, eyev, ones_v, zbuf, acc):
    core = lax.axis_index("core")
    sub = lax.axis_index("subcore")
    wid = core * 16 + sub

    @pl.loop(0, CH)
    def _(r):
        ones_v[r, pl.ds(0, 16)] = jnp.ones((16,), jnp.float32)
        zbuf[r, pl.ds(0, 16)] = jnp.zeros((16,), jnp.float32)

    pltpu.sync_copy(dst_hbm.at[wid], didx)
    pltpu.sync_copy(eye_hbm.at[sub], eyev)

    @pl.loop(0, _KCH)
    def _(k):
        pltpu.sync_copy(zbuf, acc.at[eyev.at[k]])

    plsc.subcore_barrier()

    @pl.loop(0, CPW)
    def _(c):
        pltpu.sync_copy(ones_v, acc.at[didx.at[c]], add=True)

    plsc.subcore_barrier()

    @pl.loop(0, _KCH)
    def _(k):
        pltpu.sync_copy(acc.at[eyev.at[k]], zbuf)
        pltpu.sync_copy(zbuf, degp_hbm.at[core, pl.ds(sub * RPS + k * CH, CH)])


@functools.cache
def _make_sc_scatter():
    mesh = plsc.VectorSubcoreMesh(core_axis_name="core",
                                  subcore_axis_name="subcore")
    return functools.partial(
        pl.kernel,
        out_type=jax.ShapeDtypeStruct((2, N_ACC, D), jnp.float32),
        mesh=mesh,
        scratch_types=[
            pltpu.VMEM((GRP, CH), jnp.int32),
            pltpu.VMEM((GRP, CH), jnp.int32),
            pltpu.VMEM((_KCH, CH), jnp.int32),
            pltpu.VMEM((CH, D), jnp.float32),
            pltpu.VMEM((CH, D), jnp.float32),
            pltpu.VMEM_SHARED((N_ACC, D), jnp.float32),
            pltpu.SemaphoreType.DMA,
            pltpu.SemaphoreType.DMA,
            pltpu.SemaphoreType.DMA,
        ],
    )(_sc_scatter_body)


def _sc_scatter_body(hs_hbm, src_hbm, dst_hbm, eye_hbm, out_hbm,
                     sbuf, dbuf, eyev, rows_a, rows_b, acc,
                     gsem, sem_a, sem_b):
    core = lax.axis_index("core")
    sub = lax.axis_index("subcore")
    wid = core * 16 + sub

    pltpu.sync_copy(eye_hbm.at[sub], eyev)

    @pl.loop(0, CH)
    def _(r):
        for j in range(D // 16):
            rows_a[r, pl.ds(j * 16, 16)] = jnp.zeros((16,), jnp.float32)

    @pl.loop(0, _KCH)
    def _(k):
        pltpu.sync_copy(rows_a, acc.at[eyev.at[k]])

    plsc.subcore_barrier()

    @pl.loop(0, CPW // GRP)
    def _(g):
        pltpu.sync_copy(src_hbm.at[wid, pl.ds(g * GRP, GRP)], sbuf)
        pltpu.sync_copy(dst_hbm.at[wid, pl.ds(g * GRP, GRP)], dbuf)
        for c in range(GRP):
            ga = pltpu.make_async_copy(hs_hbm.at[sbuf.at[c]], rows_a, gsem)
            ga.start()
            ga.wait()

            @pl.loop(0, CH)
            def _(r):
                for j in range(D // 16):
                    rows_b[r, pl.ds(j * 16, 16)] = rows_a[r, pl.ds(j * 16, 16)]

            s = pltpu.make_async_copy(rows_b, acc.at[dbuf.at[c]], sem_a)
            s.start(add=True)
            s.wait()

    plsc.subcore_barrier()

    @pl.loop(0, _KCH)
    def _(k):
        pltpu.sync_copy(acc.at[eyev.at[k]], rows_a)
        pltpu.sync_copy(rows_a,
                        out_hbm.at[core, pl.ds(sub * RPS + k * CH, CH)])


_RB = 1000
_NBLK = N // _RB


def _tc_matmul0(x, w):
    def body(x_ref, w_ref, o_ref):
        o_ref[...] = jnp.dot(x_ref[...], w_ref[...],
                             preferred_element_type=jnp.float32)

    return pl.pallas_call(
        body,
        grid=(_NBLK,),
        in_specs=[
            pl.BlockSpec((_RB, D), lambda i: (i, 0)),
            pl.BlockSpec((D, D), lambda i: (0, 0)),
        ],
        out_specs=pl.BlockSpec((_RB, D), lambda i: (i, 0)),
        out_shape=jax.ShapeDtypeStruct((N, D), jnp.float32),
    )(x, w)


def _tc_prescale(degp, y0):
    def body(degp_ref, y0_ref, dinv_ref, dinv2_ref, hs_ref):
        dd = degp_ref[0] + degp_ref[1]
        deg = jnp.sum(dd, axis=1, keepdims=True) + 1.0
        dinv = lax.rsqrt(deg)
        dinv_ref[...] = dinv
        dinv2_ref[...] = dinv * dinv
        hs_ref[...] = y0_ref[...] * dinv

    return pl.pallas_call(
        body,
        grid=(_NBLK,),
        in_specs=[
            pl.BlockSpec((2, _RB, 16), lambda i: (0, i, 0)),
            pl.BlockSpec((_RB, D), lambda i: (i, 0)),
        ],
        out_specs=[
            pl.BlockSpec((_RB, 1), lambda i: (i, 0)),
            pl.BlockSpec((_RB, 1), lambda i: (i, 0)),
            pl.BlockSpec((_RB, D), lambda i: (i, 0)),
        ],
        out_shape=[
            jax.ShapeDtypeStruct((N, 1), jnp.float32),
            jax.ShapeDtypeStruct((N, 1), jnp.float32),
            jax.ShapeDtypeStruct((N, D), jnp.float32),
        ],
    )(degp, y0)


def _tc_act_stats(mp, y, dinv, dinv2, b):
    def body(mp_ref, y_ref, dinv_ref, dinv2_ref, b_ref, a_ref, s1_ref, s2_ref):
        i = pl.program_id(0)
        m = dinv_ref[...] * (mp_ref[0] + mp_ref[1]) \
            + dinv2_ref[...] * y_ref[...] + b_ref[...]
        a = jnp.maximum(m, 0.0)
        a_ref[...] = a

        @pl.when(i == 0)
        def _():
            s1_ref[...] = jnp.zeros_like(s1_ref)
            s2_ref[...] = jnp.zeros_like(s2_ref)

        s1_ref[...] += jnp.sum(a, axis=0, keepdims=True)
        s2_ref[...] += jnp.sum(a * a, axis=0, keepdims=True)

    return pl.pallas_call(
        body,
        grid=(_NBLK,),
        in_specs=[
            pl.BlockSpec((2, _RB, D), lambda i: (0, i, 0)),
            pl.BlockSpec((_RB, D), lambda i: (i, 0)),
            pl.BlockSpec((_RB, 1), lambda i: (i, 0)),
            pl.BlockSpec((_RB, 1), lambda i: (i, 0)),
            pl.BlockSpec((1, D), lambda i: (0, 0)),
        ],
        out_specs=[
            pl.BlockSpec((_RB, D), lambda i: (i, 0)),
            pl.BlockSpec((1, D), lambda i: (0, 0)),
            pl.BlockSpec((1, D), lambda i: (0, 0)),
        ],
        out_shape=[
            jax.ShapeDtypeStruct((N, D), jnp.float32),
            jax.ShapeDtypeStruct((1, D), jnp.float32),
            jax.ShapeDtypeStruct((1, D), jnp.float32),
        ],
    )(mp, y, dinv, dinv2, b)


def _tc_bn_matmul(a, s1, s2, g, be, w, dinv):
    def body(a_ref, s1_ref, s2_ref, g_ref, be_ref, w_ref, dinv_ref,
             y_ref, hs_ref):
        mean = s1_ref[...] * (1.0 / N)
        var = s2_ref[...] * (1.0 / N) - mean * mean
        sc = g_ref[...] * lax.rsqrt(var + BN_EPS)
        h = (a_ref[...] - mean) * sc + be_ref[...]
        y = jnp.dot(h, w_ref[...], preferred_element_type=jnp.float32)
        y_ref[...] = y
        hs_ref[...] = y * dinv_ref[...]

    return pl.pallas_call(
        body,
        grid=(_NBLK,),
        in_specs=[
            pl.BlockSpec((_RB, D), lambda i: (i, 0)),
            pl.BlockSpec((1, D), lambda i: (0, 0)),
            pl.BlockSpec((1, D), lambda i: (0, 0)),
            pl.BlockSpec((1, D), lambda i: (0, 0)),
            pl.BlockSpec((1, D), lambda i: (0, 0)),
            pl.BlockSpec((D, D), lambda i: (0, 0)),
            pl.BlockSpec((_RB, 1), lambda i: (i, 0)),
        ],
        out_specs=[
            pl.BlockSpec((_RB, D), lambda i: (i, 0)),
            pl.BlockSpec((_RB, D), lambda i: (i, 0)),
        ],
        out_shape=[
            jax.ShapeDtypeStruct((N, D), jnp.float32),
            jax.ShapeDtypeStruct((N, D), jnp.float32),
        ],
    )(a, s1, s2, g, be, w, dinv)


def _tc_act_pool(mp, y, dinv, dinv2, b, batch2):
    def body(mp_ref, y_ref, dinv_ref, dinv2_ref, b_ref, bt_ref,
             p_ref, cnt_ref, s1_ref, s2_ref):
        i = pl.program_id(0)
        m = dinv_ref[...] * (mp_ref[0] + mp_ref[1]) \
            + dinv2_ref[...] * y_ref[...] + b_ref[...]
        a = jnp.maximum(m, 0.0)

        @pl.when(i == 0)
        def _():
            p_ref[...] = jnp.zeros_like(p_ref)
            cnt_ref[...] = jnp.zeros_like(cnt_ref)
            s1_ref[...] = jnp.zeros_like(s1_ref)
            s2_ref[...] = jnp.zeros_like(s2_ref)

        s1_ref[...] += jnp.sum(a, axis=0, keepdims=True)
        s2_ref[...] += jnp.sum(a * a, axis=0, keepdims=True)

        gid = lax.broadcasted_iota(jnp.int32, (_RB, G), 1)
        oh = jnp.where(bt_ref[...] == gid, 1.0, 0.0)
        dn = (((0,), (0,)), ((), ()))
        p_ref[...] += lax.dot_general(oh, a, dn,
                                      preferred_element_type=jnp.float32)
        ones = jnp.ones((_RB, 1), jnp.float32)
        cnt_ref[...] += lax.dot_general(oh, ones, dn,
                                        preferred_element_type=jnp.float32)

    return pl.pallas_call(
        body,
        grid=(_NBLK,),
        in_specs=[
            pl.BlockSpec((2, _RB, D), lambda i: (0, i, 0)),
            pl.BlockSpec((_RB, D), lambda i: (i, 0)),
            pl.BlockSpec((_RB, 1), lambda i: (i, 0)),
            pl.BlockSpec((_RB, 1), lambda i: (i, 0)),
            pl.BlockSpec((1, D), lambda i: (0, 0)),
            pl.BlockSpec((_RB, 1), lambda i: (i, 0)),
        ],
        out_specs=[
            pl.BlockSpec((G, D), lambda i: (0, 0)),
            pl.BlockSpec((G, 1), lambda i: (0, 0)),
            pl.BlockSpec((1, D), lambda i: (0, 0)),
            pl.BlockSpec((1, D), lambda i: (0, 0)),
        ],
        out_shape=[
            jax.ShapeDtypeStruct((G, D), jnp.float32),
            jax.ShapeDtypeStruct((G, 1), jnp.float32),
            jax.ShapeDtypeStruct((1, D), jnp.float32),
            jax.ShapeDtypeStruct((1, D), jnp.float32),
        ],
    )(mp, y, dinv, dinv2, b, batch2)


def _tc_head(p_raw, cnt, s1, s2, g1, be1, wh0, bh0, gh0, beh0,
             wh1, bh1, gh1, beh1, wf, bf):
    def body(p_ref, cnt_ref, s1_ref, s2_ref, g1_ref, be1_ref,
             wh0_ref, bh0_ref, gh0_ref, beh0_ref,
             wh1_ref, bh1_ref, gh1_ref, beh1_ref, wf_ref, bf_ref, o_ref):
        mean = s1_ref[...] * (1.0 / N)
        var = s2_ref[...] * (1.0 / N) - mean * mean
        sc = g1_ref[...] * lax.rsqrt(var + BN_EPS)
        p = sc * p_ref[...] + cnt_ref[...] * (be1_ref[...] - sc * mean)

        def mlp_block(q, w, b, gg, bb):
            q = jnp.maximum(jnp.dot(q, w, preferred_element_type=jnp.float32)
                            + b, 0.0)
            m = jnp.mean(q, axis=0, keepdims=True)
            v = jnp.mean(jnp.square(q - m), axis=0, keepdims=True)
            return (q - m) * lax.rsqrt(v + BN_EPS) * gg + bb

        q = mlp_block(p, wh0_ref[...], bh0_ref[...], gh0_ref[...],
                      beh0_ref[...])
        q = mlp_block(q, wh1_ref[...], bh1_ref[...], gh1_ref[...],
                      beh1_ref[...])
        o_ref[...] = jnp.dot(q, wf_ref[...],
                             preferred_element_type=jnp.float32) + bf_ref[...]

    full = lambda s: pl.BlockSpec(s, lambda: tuple(0 for _ in s))
    return pl.pallas_call(
        body,
        in_specs=[
            full((G, D)), full((G, 1)), full((1, D)), full((1, D)),
            full((1, D)), full((1, D)),
            full((D, D)), full((1, D)), full((1, D)), full((1, D)),
            full((D, D)), full((1, D)), full((1, D)), full((1, D)),
            full((D, 1)), full((1, 1)),
        ],
        out_specs=full((G, 1)),
        out_shape=jax.ShapeDtypeStruct((G, 1), jnp.float32),
    )(p_raw, cnt, s1, s2, g1, be1, wh0, bh0, gh0, beh0,
      wh1, bh1, gh1, beh1, wf, bf)


def kernel(x, edge_index, batch, W0, b0, g0, be0, W1, b1, g1, be1,
           Wh0, bh0, gh0, beh0, Wh1, bh1, gh1, beh1, Wf, bf):
    f32 = jnp.float32
    pad = E_PAD - E
    src3 = jnp.concatenate(
        [edge_index[0], jnp.zeros((pad,), jnp.int32)]).reshape(NW, CPW, CH)
    dst3 = jnp.concatenate(
        [edge_index[1], jnp.full((pad,), N, jnp.int32)]).reshape(NW, CPW, CH)
    batch2 = batch.reshape(N, 1)
    eye3 = jnp.arange(N_ACC, dtype=jnp.int32).reshape(16, _KCH, CH)

    row = lambda v: v.reshape(1, -1).astype(f32)

    sc_degree = _make_sc_degree()
    sc_scatter = _make_sc_scatter()

    degp = sc_degree(dst3, eye3)
    if False:
        deg = jnp.sum(degp[0] + degp[1], axis=1)[:N] + 1.0
        dinv_d = lax.rsqrt(deg)

        def conv(h, w, bb):
            hh = h @ w
            hs = hh * dinv_d[:, None]
            s = jax.ops.segment_sum(hs[edge_index[0]], edge_index[1],
                                    num_segments=N)
            return dinv_d[:, None] * s + (dinv_d * dinv_d)[:, None] * hh + bb

        def bn(v, gg, bb):
            mu = jnp.mean(v, axis=0, keepdims=True)
            va = jnp.var(v, axis=0, keepdims=True)
            return (v - mu) / jnp.sqrt(va + BN_EPS) * gg + bb

        mp_t = sc_scatter(x, src3, dst3, eye3)
        vals = (jnp.arange(E, dtype=jnp.int32) % CH).astype(f32)
        exp = jax.ops.segment_sum(vals, edge_index[1], num_segments=N)
        err = jnp.sum(jnp.square((mp_t[0] + mp_t[1])[:N] - exp[:, None]))
        h = bn(jax.nn.relu(conv(x, W0, b0)), g0, be0)
        h = bn(jax.nn.relu(conv(h, W1, b1)), g1, be1)
        p = jax.ops.segment_sum(h, batch, num_segments=G)
        p = bn(jax.nn.relu(p @ Wh0 + bh0), gh0, beh0)
        p = bn(jax.nn.relu(p @ Wh1 + bh1), gh1, beh1)
        return p @ Wf + bf + 1e3 * err
    y0 = _tc_matmul0(x, W0)
    dinv, dinv2, hs0 = _tc_prescale(degp, y0)

    m0p = sc_scatter(hs0, src3, dst3, eye3)
    a0, s1a, s2a = _tc_act_stats(m0p, y0, dinv, dinv2, row(b0))
    y1, hs1 = _tc_bn_matmul(a0, s1a, s2a, row(g0), row(be0), W1, dinv)

    m1p = sc_scatter(hs1, src3, dst3, eye3)
    P, cnt, s1b, s2b = _tc_act_pool(m1p, y1, dinv, dinv2, row(b1), batch2)

    return _tc_head(P, cnt, s1b, s2b, row(g1), row(be1),
                    Wh0, row(bh0), row(gh0), row(beh0),
                    Wh1, row(bh1), row(gh1), row(beh1),
                    Wf, bf.reshape(1, 1))
